# Initial kernel scaffold; baseline (speedup 1.0000x reference)
import jax
import jax.numpy as jnp
from jax.experimental import pallas as pl


def kernel(inputs, labels, syn0, syn1, biases):
    raise NotImplementedError("write your pallas kernel here")


# speedup vs baseline: 52.0032x; 52.0032x over previous
import functools

import numpy as np
import jax
import jax.numpy as jnp
from jax import lax
from jax.experimental import pallas as pl
from jax.experimental.pallas import tpu as pltpu
from jax.experimental.pallas import tpu_sc as plsc

_VOCAB = 1000
_HIDDEN = 300
_BATCH = 16384
_NEG = 5
_POWER = 0.75
_NCOL = _NEG + 1

_VP = 1024
_HP = 384

_NC = 2
_NS = 16
_NW = _NC * _NS
_BPW = _BATCH // _NW
_CHUNK = 128
_NCHUNK = _BPW // _CHUNK


_SAMPLED_T = None


def _sampled_t():
    global _SAMPLED_T
    if _SAMPLED_T is None:
        with jax.ensure_compile_time_eval():
            counts = jnp.ones((_VOCAB,), dtype=jnp.float32)
            logits = _POWER * jnp.log(counts)
            skey = jax.random.key(42)
            s = jax.random.categorical(skey, logits, shape=(_BATCH * _NEG,))
        _SAMPLED_T = np.ascontiguousarray(
            np.asarray(s).reshape(_BATCH, _NEG).astype(np.int32).T)
    return _SAMPLED_T


def _scores_body(a_ref, b_ref, bias_ref, o_ref):
    o_ref[...] = lax.dot_general(
        a_ref[...], b_ref[...], (((1,), (1,)), ((), ())),
        preferred_element_type=jnp.float32,
        precision=lax.Precision.HIGHEST,
    ) + bias_ref[...]


_scores = pl.pallas_call(
    _scores_body,
    out_shape=jax.ShapeDtypeStruct((_VP, _VP), jnp.float32),
)


@functools.lru_cache(maxsize=1)
def _gather_kernel():
    mesh = plsc.VectorSubcoreMesh(core_axis_name="c", subcore_axis_name="s")

    @functools.partial(
        pl.kernel,
        mesh=mesh,
        out_type=jax.ShapeDtypeStruct((_NCOL, _BATCH), jnp.float32),
        scratch_types=[
            pltpu.VMEM((_BPW,), jnp.int32),
            pltpu.VMEM((_NCOL, _BPW), jnp.int32),
            pltpu.VMEM((_NCOL, _BPW), jnp.int32),
            pltpu.VMEM((_NCOL, _BPW), jnp.float32),
            pltpu.SemaphoreType.DMA,
        ],
    )
    def _gather_k(g_hbm, inp_hbm, cols_hbm, out_hbm, inp_v, cols_v, idx_v, val_v, sem):
        wid = lax.axis_index("s") * _NC + lax.axis_index("c")
        base = wid * _BPW
        pltpu.sync_copy(inp_hbm.at[pl.ds(base, _BPW)], inp_v)
        for j in range(_NCOL):
            pltpu.sync_copy(cols_hbm.at[j, pl.ds(base, _BPW)], cols_v.at[j])

        def body(i, carry):
            s = pl.ds(i * 16, 16)
            iv = inp_v[s] * _VP
            for j in range(_NCOL):
                idx_v[j, s] = iv + cols_v[j, s]
            return carry

        lax.fori_loop(0, _BPW // 16, body, 0)

        copies = []
        for j in range(_NCOL):
            for c in range(_NCHUNK):
                s = pl.ds(c * _CHUNK, _CHUNK)
                copies.append(
                    pltpu.async_copy(g_hbm.at[idx_v.at[j, s]], val_v.at[j, s], sem))
        for cp in copies:
            cp.wait()

        for j in range(_NCOL):
            pltpu.sync_copy(val_v.at[j], out_hbm.at[j, pl.ds(base, _BPW)])

    return _gather_k


def _softplus_body(x_ref, o_ref):
    x = x_ref[...]
    row = lax.broadcasted_iota(jnp.int32, x.shape, 0)
    x = jnp.where(row == 0, -x, x)
    o_ref[...] = jnp.maximum(x, 0.0) + jnp.log1p(jnp.exp(-jnp.abs(x)))


_softplus = pl.pallas_call(
    _softplus_body,
    out_shape=jax.ShapeDtypeStruct((_NCOL, _BATCH), jnp.float32),
)


def kernel(inputs, labels, syn0, syn1, biases):
    syn0p = jnp.zeros((_VP, _HP), jnp.float32).at[:_VOCAB, :_HIDDEN].set(syn0)
    syn1p = jnp.zeros((_VP, _HP), jnp.float32).at[:_VOCAB, :_HIDDEN].set(syn1)
    biasp = jnp.zeros((1, _VP), jnp.float32).at[0, :_VOCAB].set(biases)
    g = _scores(syn0p, syn1p, biasp).reshape(_VP * _VP)
    cols = jnp.concatenate(
        [labels[None, :].astype(jnp.int32), jnp.asarray(_sampled_t())], axis=0)
    logits_t = _gather_kernel()(g, inputs.astype(jnp.int32), cols)
    return _softplus(logits_t).T
